# baseline (speedup 1.0000x reference)
import numpy as np

import jax
import jax.numpy as jnp
from jax.experimental import pallas as pl
from jax.experimental.pallas import tpu as pltpu

_TB = 1024


def _moe_block_kernel(x_ref, wg_ref, bg_ref, w1_ref, b1_ref, w2_ref, b2_ref,
                      s_ref, r_ref, out_ref, w1s_ref, w2s_ref):
    n_experts, _, d_hidden = w1_ref.shape
    n_act = out_ref.shape[-1]

    @pl.when(pl.program_id(0) == 0)
    def _fill_weight_scratch():
        for e in range(n_experts):
            w1s_ref[:, e * d_hidden:(e + 1) * d_hidden] = (
                w1_ref[e].astype(jnp.bfloat16))
        w2s_ref[...] = jnp.zeros(w2s_ref.shape, jnp.bfloat16)
        for e in range(n_experts):
            w2s_ref[e * d_hidden:(e + 1) * d_hidden,
                    e * n_act:(e + 1) * n_act] = w2_ref[e].astype(jnp.bfloat16)

    x = x_ref[...].astype(jnp.bfloat16)

    logits = jnp.dot(x, wg_ref[...], preferred_element_type=jnp.float32)
    logits = logits + bg_ref[...]
    expg = jnp.exp(logits - jnp.max(logits, axis=-1, keepdims=True))
    denom = jnp.sum(expg, axis=-1, keepdims=True)
    expg16 = expg.astype(jnp.bfloat16)
    gate_a = jnp.dot(expg16, s_ref[...],
                     preferred_element_type=jnp.float32)

    h = jnp.dot(x, w1s_ref[...], preferred_element_type=jnp.float32)
    hr = jnp.maximum(h + b1_ref[...], 0.0).astype(jnp.bfloat16)

    o_all = jnp.dot(hr, w2s_ref[...],
                    preferred_element_type=jnp.float32)
    og = (o_all * gate_a).astype(jnp.bfloat16)
    acc = jnp.dot(og, r_ref[...], preferred_element_type=jnp.float32)
    acc = acc + jnp.dot(expg16, b2_ref[...],
                        preferred_element_type=jnp.float32)
    out_ref[...] = acc / denom


def kernel(features, Wg, bg, W1, b1, W2, b2):
    t, d = features.shape
    e, _, h = W1.shape
    a = W2.shape[-1]

    wgb = Wg.astype(jnp.bfloat16)
    bg2 = bg.reshape(1, e)
    b1c = b1.reshape(1, e * h)
    s = jnp.asarray(np.kron(np.eye(e, dtype=np.float32),
                            np.ones((1, a), np.float32)), dtype=jnp.bfloat16)
    r = jnp.asarray(np.kron(np.ones((e, 1), np.float32),
                            np.eye(a, dtype=np.float32)), dtype=jnp.bfloat16)

    grid = (t // _TB,)
    return pl.pallas_call(
        _moe_block_kernel,
        grid=grid,
        in_specs=[
            pl.BlockSpec((_TB, d), lambda i: (i, 0)),
            pl.BlockSpec((d, e), lambda i: (0, 0)),
            pl.BlockSpec((1, e), lambda i: (0, 0)),
            pl.BlockSpec((e, d, h), lambda i: (0, 0, 0)),
            pl.BlockSpec((1, e * h), lambda i: (0, 0)),
            pl.BlockSpec((e, h, a), lambda i: (0, 0, 0)),
            pl.BlockSpec((e, a), lambda i: (0, 0)),
            pl.BlockSpec((e, e * a), lambda i: (0, 0)),
            pl.BlockSpec((e * a, a), lambda i: (0, 0)),
        ],
        out_specs=pl.BlockSpec((_TB, a), lambda i: (i, 0)),
        out_shape=jax.ShapeDtypeStruct((t, a), jnp.float32),
        scratch_shapes=[
            pltpu.VMEM((d, e * h), jnp.bfloat16),
            pltpu.VMEM((e * h, e * a), jnp.bfloat16),
        ],
    )(features, wgb, bg2, W1, b1c, W2, b2, s, r)


# speedup vs baseline: 1.1426x; 1.1426x over previous
import numpy as np

import jax
import jax.numpy as jnp
from jax.experimental import pallas as pl
from jax.experimental.pallas import tpu as pltpu

_TB = 1024


def _moe_block_kernel(x_ref, wg_ref, bg_ref, w1_ref, b1_ref, w2_ref, b2_ref,
                      s_ref, out_ref, w1s_ref, w2s_ref):
    n_experts, _, d_hidden = w1_ref.shape
    n_act = out_ref.shape[-1]

    @pl.when(pl.program_id(0) == 0)
    def _fill_weight_scratch():
        for e in range(n_experts):
            w1s_ref[:, e * d_hidden:(e + 1) * d_hidden] = (
                w1_ref[e].astype(jnp.bfloat16))
            w2s_ref[e * d_hidden:(e + 1) * d_hidden, :] = (
                w2_ref[e].astype(jnp.bfloat16))

    x = x_ref[...].astype(jnp.bfloat16)

    logits = jnp.dot(x, wg_ref[...], preferred_element_type=jnp.float32)
    logits = logits + bg_ref[...]
    expg = jnp.exp(logits - jnp.max(logits, axis=-1, keepdims=True))
    denom = jnp.sum(expg, axis=-1, keepdims=True)
    expg16 = expg.astype(jnp.bfloat16)
    gate_a = jnp.dot(expg16, s_ref[...],
                     preferred_element_type=jnp.float32)

    h = jnp.dot(x, w1s_ref[...], preferred_element_type=jnp.float32)
    hr = jnp.maximum(h + b1_ref[...], 0.0).astype(jnp.bfloat16)

    acc = jnp.dot(expg16, b2_ref[...], preferred_element_type=jnp.float32)
    for e in range(n_experts):
        oe = jnp.dot(hr[:, e * d_hidden:(e + 1) * d_hidden],
                     w2s_ref[e * d_hidden:(e + 1) * d_hidden, :],
                     preferred_element_type=jnp.float32)
        acc = acc + oe * gate_a[:, e * n_act:(e + 1) * n_act]
    out_ref[...] = acc / denom


def kernel(features, Wg, bg, W1, b1, W2, b2):
    t, d = features.shape
    e, _, h = W1.shape
    a = W2.shape[-1]

    wgb = Wg.astype(jnp.bfloat16)
    bg2 = bg.reshape(1, e)
    b1c = b1.reshape(1, e * h)
    s = jnp.asarray(np.kron(np.eye(e, dtype=np.float32),
                            np.ones((1, a), np.float32)), dtype=jnp.bfloat16)

    grid = (t // _TB,)
    return pl.pallas_call(
        _moe_block_kernel,
        grid=grid,
        in_specs=[
            pl.BlockSpec((_TB, d), lambda i: (i, 0)),
            pl.BlockSpec((d, e), lambda i: (0, 0)),
            pl.BlockSpec((1, e), lambda i: (0, 0)),
            pl.BlockSpec((e, d, h), lambda i: (0, 0, 0)),
            pl.BlockSpec((1, e * h), lambda i: (0, 0)),
            pl.BlockSpec((e, h, a), lambda i: (0, 0, 0)),
            pl.BlockSpec((e, a), lambda i: (0, 0)),
            pl.BlockSpec((e, e * a), lambda i: (0, 0)),
        ],
        out_specs=pl.BlockSpec((_TB, a), lambda i: (i, 0)),
        out_shape=jax.ShapeDtypeStruct((t, a), jnp.float32),
        scratch_shapes=[
            pltpu.VMEM((d, e * h), jnp.bfloat16),
            pltpu.VMEM((e * h, a), jnp.bfloat16),
        ],
    )(features, wgb, bg2, W1, b1c, W2, b2, s)
